# Initial kernel scaffold; baseline (speedup 1.0000x reference)
import jax
import jax.numpy as jnp
from jax.experimental import pallas as pl


def kernel(q_pts, s_pts, s_feats, neighb_inds, weights, kernel_points):
    raise NotImplementedError("write your pallas kernel here")


# speedup vs baseline: 1.0185x; 1.0185x over previous
import functools

import jax
import jax.numpy as jnp
from jax import lax
from jax.experimental import pallas as pl
from jax.experimental.pallas import tpu as pltpu
from jax.experimental.pallas import tpu_sc as plsc

_M = 10000
_N = 10000
_H = 32
_C = 128
_K = 15
_SIGMA = 1.0

_NC = 2
_NS = 16
_NW = _NC * _NS
_B = _M * _H
_BPW = _B // _NW
_CH = 400
_NCHUNK = _BPW // _CH


def _sc_gather(feats_tbl, pos_tbl, idx):
    mesh = plsc.VectorSubcoreMesh(core_axis_name="c", subcore_axis_name="s")

    @functools.partial(
        pl.kernel,
        mesh=mesh,
        out_type=[
            jax.ShapeDtypeStruct((_B, _C), jnp.float32),
            jax.ShapeDtypeStruct((_B, _C), jnp.float32),
        ],
        scratch_types=[
            pltpu.VMEM((_BPW,), jnp.int32),
            pltpu.VMEM((_CH, _C), jnp.float32),
            pltpu.VMEM((_CH, _C), jnp.float32),
            pltpu.SemaphoreType.DMA,
            pltpu.SemaphoreType.DMA,
        ],
    )
    def gather_kernel(feats_hbm, pos_hbm, idx_hbm, nf_out, np_out,
                      idx_v, rows_v, prow_v, sem_f, sem_p):
        wid = lax.axis_index("s") * _NC + lax.axis_index("c")
        base = wid * _BPW
        pltpu.sync_copy(idx_hbm.at[pl.ds(base, _BPW)], idx_v)
        for j in range(_NCHUNK):
            idx_chunk = idx_v.at[pl.ds(j * _CH, _CH)]
            cp_f = pltpu.async_copy(feats_hbm.at[idx_chunk], rows_v, sem_f)
            cp_p = pltpu.async_copy(pos_hbm.at[idx_chunk], prow_v, sem_p)
            cp_f.wait()
            cp_p.wait()
            pltpu.sync_copy(rows_v, nf_out.at[pl.ds(base + j * _CH, _CH)])
            pltpu.sync_copy(prow_v, np_out.at[pl.ds(base + j * _CH, _CH)])

    return gather_kernel(feats_tbl, pos_tbl, idx)


def _tc_body(nf_ref, np_ref, q_ref, kp_ref, w_ref, out_ref):
    nf = nf_ref[...]
    rel = np_ref[:, :, 0:16] - q_ref[...][:, None, :]
    kp = kp_ref[...]
    parts = []
    for k in range(_K):
        d = rel - kp[k:k + 1][None]
        d2 = jnp.sum(d * d, axis=2)
        wk = jnp.maximum(1.0 - jnp.sqrt(d2) / _SIGMA, 0.0)
        parts.append(jnp.sum(wk[:, :, None] * nf, axis=1))
    v = jnp.concatenate(parts, axis=1)
    out_ref[...] = jnp.dot(v, w_ref[...], preferred_element_type=jnp.float32)


def _tc_compute(nf, npos, q_pad, kp_pad, w_flat, tile_m=200):
    grid = (_M // tile_m,)
    return pl.pallas_call(
        _tc_body,
        grid=grid,
        in_specs=[
            pl.BlockSpec((tile_m, _H, _C), lambda i: (i, 0, 0)),
            pl.BlockSpec((tile_m, _H, _C), lambda i: (i, 0, 0)),
            pl.BlockSpec((tile_m, 16), lambda i: (i, 0)),
            pl.BlockSpec((16, 16), lambda i: (0, 0)),
            pl.BlockSpec((_K * _C, _C), lambda i: (0, 0)),
        ],
        out_specs=pl.BlockSpec((tile_m, _C), lambda i: (i, 0)),
        out_shape=jax.ShapeDtypeStruct((_M, _C), jnp.float32),
    )(nf, npos, q_pad, kp_pad, w_flat)


def kernel(q_pts, s_pts, s_feats, neighb_inds, weights, kernel_points):
    idx = neighb_inds.astype(jnp.int32).reshape(_B)
    pos_tbl = jnp.pad(s_pts, ((0, 0), (0, _C - 3)))
    nf_flat, np_flat = _sc_gather(s_feats, pos_tbl, idx)
    nf = nf_flat.reshape(_M, _H, _C)
    npos = np_flat.reshape(_M, _H, _C)
    q_pad = jnp.pad(q_pts, ((0, 0), (0, 13)))
    kp_pad = jnp.pad(kernel_points, ((0, 1), (0, 13)))
    w_flat = weights.reshape(_K * _C, _C)
    return _tc_compute(nf, npos, q_pad, kp_pad, w_flat)


# speedup vs baseline: 3.5710x; 3.5060x over previous
import functools

import jax
import jax.numpy as jnp
from jax import lax
from jax.experimental import pallas as pl
from jax.experimental.pallas import tpu as pltpu
from jax.experimental.pallas import tpu_sc as plsc

_M = 10000
_N = 10000
_H = 32
_C = 128
_K = 15
_SIGMA = 1.0

_NC = 2
_NS = 16
_NW = _NC * _NS
_B = _M * _H
_BPW = _B // _NW
_CH = 400
_NCHUNK = _BPW // _CH


def _sc_gather(feats_tbl, pos_tbl, idx):
    mesh = plsc.VectorSubcoreMesh(core_axis_name="c", subcore_axis_name="s")

    @functools.partial(
        pl.kernel,
        mesh=mesh,
        out_type=[
            jax.ShapeDtypeStruct((_B, _C), jnp.float32),
            jax.ShapeDtypeStruct((_B, _C), jnp.float32),
        ],
        scratch_types=[
            pltpu.VMEM((_BPW,), jnp.int32),
            pltpu.VMEM((_CH, _C), jnp.float32),
            pltpu.VMEM((_CH, _C), jnp.float32),
            pltpu.SemaphoreType.DMA,
            pltpu.SemaphoreType.DMA,
        ],
    )
    def gather_kernel(feats_hbm, pos_hbm, idx_hbm, nf_out, np_out,
                      idx_v, rows_v, prow_v, sem_f, sem_p):
        wid = lax.axis_index("s") * _NC + lax.axis_index("c")
        base = wid * _BPW
        pltpu.sync_copy(idx_hbm.at[pl.ds(base, _BPW)], idx_v)
        for j in range(_NCHUNK):
            idx_chunk = idx_v.at[pl.ds(j * _CH, _CH)]
            cp_f = pltpu.async_copy(feats_hbm.at[idx_chunk], rows_v, sem_f)
            cp_p = pltpu.async_copy(pos_hbm.at[idx_chunk], prow_v, sem_p)
            cp_f.wait()
            cp_p.wait()
            pltpu.sync_copy(rows_v, nf_out.at[pl.ds(base + j * _CH, _CH)])
            pltpu.sync_copy(prow_v, np_out.at[pl.ds(base + j * _CH, _CH)])

    return gather_kernel(feats_tbl, pos_tbl, idx)


def _tc_body(nf_ref, np_ref, q_ref, g_ref, w_ref, out_ref):
    nf = nf_ref[...]
    rel = np_ref[:, :, 0:16] - q_ref[...][:, None, :]
    r2 = jnp.sum(rel * rel, axis=2, keepdims=True)
    rel_h = jnp.concatenate(
        [rel[:, :, 0:3], r2, jnp.ones_like(r2)], axis=2)
    d2 = jax.lax.dot_general(
        rel_h, g_ref[0:5, :],
        dimension_numbers=(((2,), (0,)), ((), ())),
        preferred_element_type=jnp.float32)
    w_all = jnp.maximum(1.0 - jnp.sqrt(jnp.maximum(d2, 0.0)) / _SIGMA, 0.0)
    w_t = jnp.swapaxes(w_all, 1, 2)
    wf = jax.lax.dot_general(
        w_t, nf,
        dimension_numbers=(((2,), (1,)), ((0,), (0,))),
        preferred_element_type=jnp.float32)
    acc = jnp.zeros((nf.shape[0], _C), jnp.float32)
    for k in range(_K):
        acc = acc + jnp.dot(wf[:, k, :], w_ref[k],
                            preferred_element_type=jnp.float32)
    out_ref[...] = acc


def _tc_compute(nf, npos, q_pad, g_basis, weights, tile_m=200):
    grid = (_M // tile_m,)
    return pl.pallas_call(
        _tc_body,
        grid=grid,
        in_specs=[
            pl.BlockSpec((tile_m, _H, _C), lambda i: (i, 0, 0)),
            pl.BlockSpec((tile_m, _H, _C), lambda i: (i, 0, 0)),
            pl.BlockSpec((tile_m, 16), lambda i: (i, 0)),
            pl.BlockSpec((8, 16), lambda i: (0, 0)),
            pl.BlockSpec((_K, _C, _C), lambda i: (0, 0, 0)),
        ],
        out_specs=pl.BlockSpec((tile_m, _C), lambda i: (i, 0)),
        out_shape=jax.ShapeDtypeStruct((_M, _C), jnp.float32),
    )(nf, npos, q_pad, g_basis, weights)


def kernel(q_pts, s_pts, s_feats, neighb_inds, weights, kernel_points):
    idx = neighb_inds.astype(jnp.int32).reshape(_B)
    pos_tbl = jnp.pad(s_pts, ((0, 0), (0, _C - 3)))
    nf_flat, np_flat = _sc_gather(s_feats, pos_tbl, idx)
    nf = nf_flat.reshape(_M, _H, _C)
    npos = np_flat.reshape(_M, _H, _C)
    q_pad = jnp.pad(q_pts, ((0, 0), (0, 13)))
    g = jnp.zeros((8, 16), jnp.float32)
    g = g.at[0:3, 0:_K].set(-2.0 * kernel_points.T)
    g = g.at[3, 0:16].set(1.0)
    g = g.at[4, 0:_K].set(jnp.sum(kernel_points * kernel_points, axis=1))
    g = g.at[4, _K:16].set(1e9)
    return _tc_compute(nf, npos, q_pad, g, weights)


# speedup vs baseline: 3.6331x; 1.0174x over previous
import functools

import jax
import jax.numpy as jnp
from jax import lax
from jax.experimental import pallas as pl
from jax.experimental.pallas import tpu as pltpu
from jax.experimental.pallas import tpu_sc as plsc

_M = 10000
_N = 10000
_H = 32
_C = 128
_K = 15
_SIGMA = 1.0

_NC = 2
_NS = 16
_NW = _NC * _NS
_B = _M * _H
_BPW = _B // _NW
_CH = 200
_NCHUNK = _BPW // _CH


def _sc_gather(feats_tbl, pos_tbl, idx):
    mesh = plsc.VectorSubcoreMesh(core_axis_name="c", subcore_axis_name="s")

    @functools.partial(
        pl.kernel,
        mesh=mesh,
        out_type=[
            jax.ShapeDtypeStruct((_B, _C), jnp.float32),
            jax.ShapeDtypeStruct((_B, _C), jnp.float32),
        ],
        scratch_types=[
            pltpu.VMEM((_BPW,), jnp.int32),
            pltpu.VMEM((_CH, _C), jnp.float32),
            pltpu.VMEM((_CH, _C), jnp.float32),
            pltpu.VMEM((_CH, _C), jnp.float32),
            pltpu.VMEM((_CH, _C), jnp.float32),
            pltpu.SemaphoreType.DMA,
            pltpu.SemaphoreType.DMA,
            pltpu.SemaphoreType.DMA,
            pltpu.SemaphoreType.DMA,
            pltpu.SemaphoreType.DMA,
            pltpu.SemaphoreType.DMA,
            pltpu.SemaphoreType.DMA,
            pltpu.SemaphoreType.DMA,
        ],
    )
    def gather_kernel(feats_hbm, pos_hbm, idx_hbm, nf_out, np_out,
                      idx_v, f0, f1, p0, p1,
                      sif0, sif1, sip0, sip1, sof0, sof1, sop0, sop1):
        wid = lax.axis_index("s") * _NC + lax.axis_index("c")
        base = wid * _BPW
        pltpu.sync_copy(idx_hbm.at[pl.ds(base, _BPW)], idx_v)
        fbuf, pbuf = (f0, f1), (p0, p1)
        sif, sip = (sif0, sif1), (sip0, sip1)
        sof, sop = (sof0, sof1), (sop0, sop1)

        def start_in(j):
            b = j & 1
            ic = idx_v.at[pl.ds(j * _CH, _CH)]
            return (pltpu.async_copy(feats_hbm.at[ic], fbuf[b], sif[b]),
                    pltpu.async_copy(pos_hbm.at[ic], pbuf[b], sip[b]))

        def start_out(j):
            b = j & 1
            dst = pl.ds(base + j * _CH, _CH)
            return (pltpu.async_copy(fbuf[b], nf_out.at[dst], sof[b]),
                    pltpu.async_copy(pbuf[b], np_out.at[dst], sop[b]))

        cp_in = start_in(0)
        cp_out_prev = None
        for j in range(_NCHUNK):
            if j + 1 < _NCHUNK:
                if cp_out_prev is not None:
                    cp_out_prev[0].wait()
                    cp_out_prev[1].wait()
                cp_in_next = start_in(j + 1)
            cp_in[0].wait()
            cp_in[1].wait()
            cp_out = start_out(j)
            if j + 1 < _NCHUNK:
                cp_out_prev, cp_in = cp_out, cp_in_next
            else:
                cp_out_prev[0].wait()
                cp_out_prev[1].wait()
                cp_out[0].wait()
                cp_out[1].wait()

    return gather_kernel(feats_tbl, pos_tbl, idx)


def _tc_body(nf_ref, np_ref, q_ref, g_ref, w_ref, out_ref):
    nf = nf_ref[...]
    rel = np_ref[:, :, 0:16] - q_ref[...][:, None, :]
    r2 = jnp.sum(rel * rel, axis=2, keepdims=True)
    rel_h = jnp.concatenate(
        [rel[:, :, 0:3], r2, jnp.ones_like(r2)], axis=2)
    d2 = jax.lax.dot_general(
        rel_h, g_ref[0:5, :],
        dimension_numbers=(((2,), (0,)), ((), ())),
        preferred_element_type=jnp.float32)
    w_all = jnp.maximum(1.0 - jnp.sqrt(jnp.maximum(d2, 0.0)) / _SIGMA, 0.0)
    w_t = jnp.swapaxes(w_all, 1, 2)
    wf = jax.lax.dot_general(
        w_t, nf,
        dimension_numbers=(((2,), (1,)), ((0,), (0,))),
        preferred_element_type=jnp.float32)
    acc = jnp.zeros((nf.shape[0], _C), jnp.float32)
    for k in range(_K):
        acc = acc + jnp.dot(wf[:, k, :], w_ref[k],
                            preferred_element_type=jnp.float32)
    out_ref[...] = acc


def _tc_compute(nf, npos, q_pad, g_basis, weights, tile_m=200):
    grid = (_M // tile_m,)
    return pl.pallas_call(
        _tc_body,
        grid=grid,
        in_specs=[
            pl.BlockSpec((tile_m, _H, _C), lambda i: (i, 0, 0)),
            pl.BlockSpec((tile_m, _H, _C), lambda i: (i, 0, 0)),
            pl.BlockSpec((tile_m, 16), lambda i: (i, 0)),
            pl.BlockSpec((8, 16), lambda i: (0, 0)),
            pl.BlockSpec((_K, _C, _C), lambda i: (0, 0, 0)),
        ],
        out_specs=pl.BlockSpec((tile_m, _C), lambda i: (i, 0)),
        out_shape=jax.ShapeDtypeStruct((_M, _C), jnp.float32),
    )(nf, npos, q_pad, g_basis, weights)


def kernel(q_pts, s_pts, s_feats, neighb_inds, weights, kernel_points):
    idx = neighb_inds.astype(jnp.int32).reshape(_B)
    pos_tbl = jnp.pad(s_pts, ((0, 0), (0, _C - 3)))
    nf_flat, np_flat = _sc_gather(s_feats, pos_tbl, idx)
    nf = nf_flat.reshape(_M, _H, _C)
    npos = np_flat.reshape(_M, _H, _C)
    q_pad = jnp.pad(q_pts, ((0, 0), (0, 13)))
    g = jnp.zeros((8, 16), jnp.float32)
    g = g.at[0:3, 0:_K].set(-2.0 * kernel_points.T)
    g = g.at[3, 0:16].set(1.0)
    g = g.at[4, 0:_K].set(jnp.sum(kernel_points * kernel_points, axis=1))
    g = g.at[4, _K:16].set(1e9)
    return _tc_compute(nf, npos, q_pad, g, weights)


# speedup vs baseline: 3.7480x; 1.0316x over previous
import functools

import jax
import jax.numpy as jnp
from jax import lax
from jax.experimental import pallas as pl
from jax.experimental.pallas import tpu as pltpu
from jax.experimental.pallas import tpu_sc as plsc

_M = 10000
_N = 10000
_H = 32
_C = 128
_K = 15
_SIGMA = 1.0

_NC = 2
_NS = 16
_NW = _NC * _NS
_B = _M * _H
_BPW = _B // _NW
_CH = 200
_NCHUNK = _BPW // _CH


def _sc_gather(feats_tbl, pos_tbl, idx):
    nb = idx.shape[0]
    bpw = nb // _NW
    nchunk = bpw // _CH
    mesh = plsc.VectorSubcoreMesh(core_axis_name="c", subcore_axis_name="s")

    @functools.partial(
        pl.kernel,
        mesh=mesh,
        out_type=[
            jax.ShapeDtypeStruct((nb, _C), jnp.float32),
            jax.ShapeDtypeStruct((nb, _C), jnp.float32),
        ],
        scratch_types=[
            pltpu.VMEM((bpw,), jnp.int32),
            pltpu.VMEM((_CH, _C), jnp.float32),
            pltpu.VMEM((_CH, _C), jnp.float32),
            pltpu.VMEM((_CH, _C), jnp.float32),
            pltpu.VMEM((_CH, _C), jnp.float32),
            pltpu.SemaphoreType.DMA,
            pltpu.SemaphoreType.DMA,
            pltpu.SemaphoreType.DMA,
            pltpu.SemaphoreType.DMA,
            pltpu.SemaphoreType.DMA,
            pltpu.SemaphoreType.DMA,
            pltpu.SemaphoreType.DMA,
            pltpu.SemaphoreType.DMA,
        ],
    )
    def gather_kernel(feats_hbm, pos_hbm, idx_hbm, nf_out, np_out,
                      idx_v, f0, f1, p0, p1,
                      sif0, sif1, sip0, sip1, sof0, sof1, sop0, sop1):
        wid = lax.axis_index("s") * _NC + lax.axis_index("c")
        base = wid * bpw
        pltpu.sync_copy(idx_hbm.at[pl.ds(base, bpw)], idx_v)
        fbuf, pbuf = (f0, f1), (p0, p1)
        sif, sip = (sif0, sif1), (sip0, sip1)
        sof, sop = (sof0, sof1), (sop0, sop1)

        def start_in(j):
            b = j & 1
            ic = idx_v.at[pl.ds(j * _CH, _CH)]
            return (pltpu.async_copy(feats_hbm.at[ic], fbuf[b], sif[b]),
                    pltpu.async_copy(pos_hbm.at[ic], pbuf[b], sip[b]))

        def start_out(j):
            b = j & 1
            dst = pl.ds(base + j * _CH, _CH)
            return (pltpu.async_copy(fbuf[b], nf_out.at[dst], sof[b]),
                    pltpu.async_copy(pbuf[b], np_out.at[dst], sop[b]))

        cp_in = start_in(0)
        cp_out_prev = None
        for j in range(nchunk):
            if j + 1 < nchunk:
                if cp_out_prev is not None:
                    cp_out_prev[0].wait()
                    cp_out_prev[1].wait()
                cp_in_next = start_in(j + 1)
            cp_in[0].wait()
            cp_in[1].wait()
            cp_out = start_out(j)
            if j + 1 < nchunk:
                cp_out_prev, cp_in = cp_out, cp_in_next
            else:
                cp_out_prev[0].wait()
                cp_out_prev[1].wait()
                cp_out[0].wait()
                cp_out[1].wait()

    return gather_kernel(feats_tbl, pos_tbl, idx)


def _tc_body(nf_ref, np_ref, q_ref, g_ref, w_ref, out_ref):
    nf = nf_ref[...]
    rel = np_ref[:, :, 0:16] - q_ref[...][:, None, :]
    r2 = jnp.sum(rel * rel, axis=2, keepdims=True)
    rel_h = jnp.concatenate(
        [rel[:, :, 0:3], r2, jnp.ones_like(r2)], axis=2)
    d2 = jax.lax.dot_general(
        rel_h, g_ref[0:5, :],
        dimension_numbers=(((2,), (0,)), ((), ())),
        preferred_element_type=jnp.float32)
    w_all = jnp.maximum(1.0 - jnp.sqrt(jnp.maximum(d2, 0.0)) / _SIGMA, 0.0)
    w_t = jnp.swapaxes(w_all, 1, 2)
    wf = jax.lax.dot_general(
        w_t, nf,
        dimension_numbers=(((2,), (1,)), ((0,), (0,))),
        preferred_element_type=jnp.float32)
    acc = jnp.zeros((nf.shape[0], _C), jnp.float32)
    for k in range(_K):
        acc = acc + jnp.dot(wf[:, k, :], w_ref[k],
                            preferred_element_type=jnp.float32)
    out_ref[...] = acc


def _tc_compute(nf, npos, q_pad, g_basis, weights, tile_m=200):
    m = nf.shape[0]
    grid = (m // tile_m,)
    return pl.pallas_call(
        _tc_body,
        grid=grid,
        in_specs=[
            pl.BlockSpec((tile_m, _H, _C), lambda i: (i, 0, 0)),
            pl.BlockSpec((tile_m, _H, _C), lambda i: (i, 0, 0)),
            pl.BlockSpec((tile_m, 16), lambda i: (i, 0)),
            pl.BlockSpec((8, 16), lambda i: (0, 0)),
            pl.BlockSpec((_K, _C, _C), lambda i: (0, 0, 0)),
        ],
        out_specs=pl.BlockSpec((tile_m, _C), lambda i: (i, 0)),
        out_shape=jax.ShapeDtypeStruct((m, _C), jnp.float32),
    )(nf, npos, q_pad, g_basis, weights)


def kernel(q_pts, s_pts, s_feats, neighb_inds, weights, kernel_points):
    idx = neighb_inds.astype(jnp.int32).reshape(_B)
    pos_tbl = jnp.pad(s_pts, ((0, 0), (0, _C - 3)))
    q_pad = jnp.pad(q_pts, ((0, 0), (0, 13)))
    g = jnp.zeros((8, 16), jnp.float32)
    g = g.at[0:3, 0:_K].set(-2.0 * kernel_points.T)
    g = g.at[3, 0:16].set(1.0)
    g = g.at[4, 0:_K].set(jnp.sum(kernel_points * kernel_points, axis=1))
    g = g.at[4, _K:16].set(1e9)
    m2 = _M // 2
    outs = []
    for h in range(2):
        idx_h = lax.slice_in_dim(idx, h * m2 * _H, (h + 1) * m2 * _H)
        nf_flat, np_flat = _sc_gather(s_feats, pos_tbl, idx_h)
        nf = nf_flat.reshape(m2, _H, _C)
        npos = np_flat.reshape(m2, _H, _C)
        q_h = lax.slice_in_dim(q_pad, h * m2, (h + 1) * m2)
        outs.append(_tc_compute(nf, npos, q_h, g, weights))
    return jnp.concatenate(outs, axis=0)
